# baseline (speedup 1.0000x reference)
import functools

import jax
import jax.numpy as jnp
from jax import lax
from jax.experimental import pallas as pl
from jax.experimental.pallas import tpu as pltpu
from jax.experimental.pallas import tpu_sc as plsc

_NC = 2
_NS = 16
_NW = _NC * _NS
_L = 16
_NBUF = 8


@functools.lru_cache(maxsize=None)
def _build(B, NUMF, CATF, V, D):
    NTOK = 1 + NUMF + CATF
    BPW = B // _NW
    SUB = 16
    NSUB = BPW // SUB
    ND = D // _L
    NBUF = _NBUF
    CHALF = CATF // 2

    mesh = plsc.VectorSubcoreMesh(core_axis_name="c", subcore_axis_name="s")

    @functools.partial(
        pl.kernel,
        out_type=jax.ShapeDtypeStruct((B, NTOK, D), jnp.float32),
        mesh=mesh,
        compiler_params=pltpu.CompilerParams(use_tc_tiling_on_sc=False),
        scratch_types=[
            pltpu.VMEM((NUMF, D), jnp.float32),
            pltpu.VMEM((NUMF, D), jnp.float32),
            pltpu.VMEM((D,), jnp.float32),
            pltpu.VMEM((NUMF, BPW), jnp.float32),
            pltpu.VMEM((SUB, 1 + NUMF, D), jnp.float32),
            pltpu.VMEM((CATF, BPW), jnp.int32),
            pltpu.VMEM((NBUF, BPW, D), jnp.float32),
            pltpu.SemaphoreType.DMA,
            pltpu.SemaphoreType.DMA,
        ],
    )
    def sc_kernel(tabA_hbm, tabB_hbm, catT_hbm, numT_hbm, w_hbm, bias_hbm,
                  cls_hbm, out_hbm, wv, bv, clsv, numv, buf, idxm, rows,
                  gsem, wsem):
        wid = lax.axis_index("s") * _NC + lax.axis_index("c")
        base = wid * BPW

        pltpu.sync_copy(catT_hbm.at[:, pl.ds(base, BPW)], idxm)
        for c in range(CATF):
            off = (c % CHALF) * V
            for p in range(BPW // _L):
                sl = pl.ds(p * _L, _L)
                idxm[c, sl] = idxm[c, sl] + off

        gathers = {}

        def start_gather(c):
            src = tabA_hbm if c < CHALF else tabB_hbm
            gathers[c] = pltpu.async_copy(
                src.at[idxm.at[c]], rows.at[c % NBUF], gsem)

        for c in range(NBUF - 1):
            start_gather(c)

        pltpu.sync_copy(w_hbm, wv)
        pltpu.sync_copy(bias_hbm, bv)
        pltpu.sync_copy(cls_hbm, clsv)
        pltpu.sync_copy(numT_hbm.at[:, pl.ds(base, BPW)], numv)

        for bl in range(SUB):
            for dd in range(ND):
                sl = pl.ds(dd * _L, _L)
                buf[bl, 0, sl] = clsv[sl]

        def num_body(s, carry):
            for i in range(NUMF):
                vec = numv[i, pl.ds(s * SUB, SUB)]
                for bl in range(SUB):
                    x = vec[bl]
                    for dd in range(ND):
                        sl = pl.ds(dd * _L, _L)
                        buf[bl, 1 + i, sl] = wv[i, sl] * x + bv[i, sl]
            pltpu.sync_copy(
                buf, out_hbm.at[pl.ds(base + s * SUB, SUB), pl.ds(0, 1 + NUMF), :])
            return carry

        lax.fori_loop(0, NSUB, num_body, 0)

        cat_writes = {}
        waited = set()
        for c in range(CATF):
            j = c + NBUF - 1
            if j < CATF:
                if c > 0:
                    cat_writes[c - 1].wait()
                    waited.add(c - 1)
                start_gather(j)
            gathers[c].wait()
            cat_writes[c] = pltpu.async_copy(
                rows.at[c % NBUF],
                out_hbm.at[pl.ds(base, BPW), 1 + NUMF + c, :], wsem)
        for c in range(CATF):
            if c not in waited:
                cat_writes[c].wait()

    return sc_kernel


def kernel(numerical, categorical, W_num, b_num, tables, cls_token):
    B, NUMF = numerical.shape
    CATF = categorical.shape[1]
    V, D = tables.shape[1], tables.shape[2]
    chalf = CATF // 2
    tab_a = tables[:chalf].reshape(chalf * V, D)
    tab_b = tables[chalf:].reshape((CATF - chalf) * V, D)
    cat_t = categorical.T.astype(jnp.int32)
    num_t = numerical.T
    cls_vec = cls_token.reshape(D)
    fn = _build(B, NUMF, CATF, V, D)
    return fn(tab_a, tab_b, cat_t, num_t, W_num, b_num, cls_vec)


# speedup vs baseline: 1.6527x; 1.6527x over previous
import functools

import jax
import jax.numpy as jnp
from jax import lax
from jax.experimental import pallas as pl
from jax.experimental.pallas import tpu as pltpu
from jax.experimental.pallas import tpu_sc as plsc

_NC = 2
_NS = 16
_NW = _NC * _NS
_L = 16


@functools.lru_cache(maxsize=None)
def _build(B, NUMF, CATF, V, D):
    NTOK = 1 + NUMF + CATF
    NPAIR = NTOK // 2
    NNUM = (1 + NUMF) // 2
    NCAT = CATF // 2
    D2 = 2 * D
    BPW = B // _NW
    SUB = 16
    NSUB = BPW // SUB
    ND = D // _L
    NB = BPW // _L

    mesh = plsc.VectorSubcoreMesh(core_axis_name="c", subcore_axis_name="s")

    @functools.partial(
        pl.kernel,
        out_type=jax.ShapeDtypeStruct((B * NPAIR, D2), jnp.float32),
        mesh=mesh,
        scratch_types=[
            pltpu.VMEM((NUMF, D), jnp.float32),
            pltpu.VMEM((NUMF, D), jnp.float32),
            pltpu.VMEM((D,), jnp.float32),
            pltpu.VMEM((NUMF, BPW), jnp.float32),
            pltpu.VMEM((CATF, BPW), jnp.int32),
            pltpu.VMEM((CATF, BPW), jnp.int32),
            pltpu.VMEM((4, BPW, D2), jnp.float32),
            pltpu.VMEM((BPW, D2), jnp.float32),
            pltpu.VMEM((NNUM * SUB, D2), jnp.float32),
            pltpu.VMEM((1, BPW), jnp.int32),
            pltpu.VMEM((1, NNUM * SUB), jnp.int32),
            pltpu.SemaphoreType.DMA,
            pltpu.SemaphoreType.DMA,
        ],
    )
    def sc_kernel(tab_hbm, catT_hbm, numT_hbm, w_hbm, bias_hbm, cls_hbm,
                  out_hbm, wv, bv, clsv, numv, idxm, idxg, ring, catbuf,
                  numbuf, didx, nidx, gsem, wsem):
        wid = lax.axis_index("s") * _NC + lax.axis_index("c")
        base = wid * BPW
        iota = lax.broadcasted_iota(jnp.int32, (_L,), 0)

        pltpu.sync_copy(catT_hbm.at[:, pl.ds(base, BPW)], idxm)
        for c in range(CATF):
            off = c * V
            for p in range(NB):
                sl = pl.ds(p * _L, _L)
                idxg[c, sl] = idxm[c, sl] + off

        def start_gather(c, slot):
            return pltpu.async_copy(
                tab_hbm.at[idxg.at[c]], ring.at[slot], gsem)

        g0 = start_gather(0, 0)
        g1 = start_gather(1, 1)

        pltpu.sync_copy(w_hbm, wv)
        pltpu.sync_copy(bias_hbm, bv)
        pltpu.sync_copy(cls_hbm, clsv)
        pltpu.sync_copy(numT_hbm.at[:, pl.ds(base, BPW)], numv)

        for j in range(SUB):
            for dd in range(ND):
                sl = pl.ds(dd * _L, _L)
                numbuf[j, sl] = clsv[sl]

        def num_body(s, carry):
            for i in range(NUMF):
                t = 1 + i
                roff = (t // 2) * SUB
                hoff = (t % 2) * D
                vec = numv[i, pl.ds(s * SUB, SUB)]
                for j in range(SUB):
                    x = vec[j]
                    for dd in range(ND):
                        numbuf[roff + j, pl.ds(hoff + dd * _L, _L)] = (
                            wv[i, pl.ds(dd * _L, _L)] * x
                            + bv[i, pl.ds(dd * _L, _L)])
            for p in range(NNUM):
                b0 = (base + s * SUB) * NPAIR + p
                nidx[0, pl.ds(p * SUB, SUB)] = iota * NPAIR + b0
            pltpu.async_copy(numbuf, out_hbm.at[nidx.at[0]], wsem).wait()
            return carry

        lax.fori_loop(0, NSUB, num_body, 0)

        def cat_body(q, carry):
            c0 = 2 * q
            slot = lax.rem(q, 2) * 2

            @pl.when(q == 0)
            def _w0():
                g0.wait()
                g1.wait()

            @pl.when(q > 0)
            def _w1():
                for _ in range(2):
                    pltpu.make_async_copy(
                        tab_hbm.at[pl.ds(0, BPW)], ring.at[0], gsem).wait()

            @pl.when(q + 1 < NCAT)
            def _prefetch():
                nslot = lax.rem(q + 1, 2) * 2
                pltpu.async_copy(
                    tab_hbm.at[idxg.at[2 * q + 2]], ring.at[nslot], gsem)
                pltpu.async_copy(
                    tab_hbm.at[idxg.at[2 * q + 3]], ring.at[nslot + 1], gsem)

            def blk_body(jj, c2):
                for dd in range(ND):
                    catbuf[jj, pl.ds(dd * _L, _L)] = ring[
                        slot, jj, pl.ds(dd * _L, _L)]
                    catbuf[jj, pl.ds(D + dd * _L, _L)] = ring[
                        slot + 1, jj, pl.ds(dd * _L, _L)]
                return c2

            lax.fori_loop(0, BPW, blk_body, 0)

            for p in range(NB):
                sl = pl.ds(p * _L, _L)
                didx[0, sl] = iota * NPAIR + ((base + p * _L) * NPAIR + NNUM + q)
            pltpu.async_copy(catbuf, out_hbm.at[didx.at[0]], wsem).wait()
            return carry

        lax.fori_loop(0, NCAT, cat_body, 0)

    return sc_kernel


def kernel(numerical, categorical, W_num, b_num, tables, cls_token):
    B, NUMF = numerical.shape
    CATF = categorical.shape[1]
    V, D = tables.shape[1], tables.shape[2]
    NTOK = 1 + NUMF + CATF
    tab_pair = jnp.pad(tables.reshape(CATF * V, D), ((0, 0), (0, D)))
    cat_t = categorical.T.astype(jnp.int32)
    num_t = numerical.T
    cls_vec = cls_token.reshape(D)
    fn = _build(B, NUMF, CATF, V, D)
    out_pair = fn(tab_pair, cat_t, num_t, W_num, b_num, cls_vec)
    return out_pair.reshape(B, NTOK, D)
